# baseline (speedup 1.0000x reference)
import functools

import jax
import jax.numpy as jnp
from jax import lax
from jax.experimental import pallas as pl
from jax.experimental.pallas import tpu as pltpu
from jax.experimental.pallas import tpu_sc as plsc

N = 10000
E = 320000
D = 128
H = 64
O = 32
BN_EPS = 1e-5

NC, NS = 2, 16
NW = NC * NS
CH = 125
K = E // (NW * CH)
RPW = N // NS
CB = 1000
RB = 2000


def _make_seg_sum(width: int, with_cnt: bool, nb: int):
  mesh = plsc.VectorSubcoreMesh(core_axis_name="c", subcore_axis_name="s")
  out_type = [jax.ShapeDtypeStruct((NC, N, width), jnp.float32)]
  if with_cnt:
    out_type.append(jax.ShapeDtypeStruct((NC, N), jnp.float32))
  scratch = [
      pltpu.VMEM((K, CH), jnp.int32),
      pltpu.VMEM((K, CH), jnp.int32),
  ] + [pltpu.VMEM((CH, width), jnp.float32) for _ in range(nb)]
  if with_cnt:
    scratch += [
        pltpu.VMEM((CH,), jnp.float32),
        pltpu.VMEM((CB,), jnp.float32),
    ]
  scratch += [
      pltpu.VMEM_SHARED((N, width), jnp.float32),
      pltpu.VMEM_SHARED((N, width), jnp.float32),
  ]
  if with_cnt:
    scratch += [pltpu.VMEM_SHARED((N,), jnp.float32)]
  scratch += [pltpu.SemaphoreType.DMA for _ in range(2 * nb)]

  def body(table, srcw, dstw, ones, zrows, zvec, *rest):
    if with_cnt:
      (parts, cnts, src_v, dst_v, *rest2) = rest
      rows = tuple(rest2[:nb])
      ones_v, zv_v, acc, tbl_sh, cacc = rest2[nb:nb + 5]
      sems = rest2[nb + 5:]
    else:
      (parts, src_v, dst_v, *rest2) = rest
      cnts = None
      rows = tuple(rest2[:nb])
      acc, tbl_sh = rest2[nb:nb + 2]
      ones_v = zv_v = cacc = None
      sems = rest2[nb + 2:]
    gsems = tuple(sems[:nb])
    ssems = tuple(sems[nb:2 * nb])
    rows0 = rows[0]
    sid = lax.axis_index("s")
    cid = lax.axis_index("c")
    wid = sid * NC + cid
    r0 = sid * RPW

    pltpu.sync_copy(zrows, rows0)
    if with_cnt:
      pltpu.sync_copy(zvec, zv_v)
    for t in range(RPW // CH):
      base = r0 + t * CH
      pltpu.sync_copy(rows0, acc.at[pl.ds(base, CH)])
    if with_cnt:
      @pl.when(sid < N // CB)
      def _zero_cnt():
        pltpu.sync_copy(zv_v, cacc.at[pl.ds(sid * CB, CB)])
      pltpu.sync_copy(ones, ones_v)
    pltpu.sync_copy(srcw.at[wid], src_v)
    pltpu.sync_copy(dstw.at[wid], dst_v)
    pltpu.sync_copy(table.at[pl.ds(r0, RPW)], tbl_sh.at[pl.ds(r0, RPW)])
    plsc.subcore_barrier()

    pltpu.async_copy(tbl_sh.at[src_v.at[0]], rows[0], gsems[0])
    pltpu.async_copy(tbl_sh.at[src_v.at[1]], rows[1], gsems[1])

    @pl.loop(0, K, step=nb)
    def _edge_ring(j):
      for b in range(nb):
        jj = j + b
        b2 = (b + 2) % nb
        pltpu.make_async_copy(tbl_sh.at[src_v.at[jj]], rows[b], gsems[b]).wait()
        pltpu.async_copy(rows[b], acc.at[dst_v.at[jj]], ssems[b], add=True)
        if with_cnt:
          pltpu.async_copy(ones_v, cacc.at[dst_v.at[jj]], ssems[b], add=True)

        @pl.when(jj + 2 < K)
        def _prefetch():
          prev = jj + 2 - nb

          @pl.when(prev >= 0)
          def _drain_scatter():
            pltpu.make_async_copy(
                rows[b2], acc.at[dst_v.at[prev]], ssems[b2]).wait()
            if with_cnt:
              pltpu.make_async_copy(
                  ones_v, cacc.at[dst_v.at[prev]], ssems[b2]).wait()
          pltpu.async_copy(tbl_sh.at[src_v.at[jj + 2]], rows[b2], gsems[b2])

    for c in range(K - nb, K):
      b = c % nb
      pltpu.make_async_copy(rows[b], acc.at[dst_v.at[c]], ssems[b]).wait()
      if with_cnt:
        pltpu.make_async_copy(ones_v, cacc.at[dst_v.at[c]], ssems[b]).wait()

    plsc.subcore_barrier()

    for t in range(RPW // CH):
      base = r0 + t * CH
      pltpu.sync_copy(acc.at[pl.ds(base, CH)], rows0)
      pltpu.sync_copy(rows0, parts.at[cid, pl.ds(base, CH)])
    if with_cnt:
      @pl.when(sid < N // CB)
      def _write_cnt():
        pltpu.sync_copy(cacc.at[pl.ds(sid * CB, CB)], zv_v)
        pltpu.sync_copy(zv_v, cnts.at[cid, pl.ds(sid * CB, CB)])

  return pl.kernel(body, out_type=tuple(out_type), mesh=mesh,
                   scratch_types=scratch,
                   compiler_params=pltpu.CompilerParams(
                       use_tc_tiling_on_sc=False))


_seg_sum_cnt = _make_seg_sum(H, with_cnt=True, nb=2)
_seg_sum_o = _make_seg_sum(O, with_cnt=False, nb=2)


def _tc_project(x, wl, wr):
  def body(x_ref, wl_ref, wr_ref, p_ref, r_ref):
    xb = x_ref[...]
    p_ref[...] = jnp.dot(xb, wl_ref[...], preferred_element_type=jnp.float32)
    r_ref[...] = jnp.dot(xb, wr_ref[...], preferred_element_type=jnp.float32)

  return pl.pallas_call(
      body,
      grid=(N // RB,),
      in_specs=[
          pl.BlockSpec((RB, D), lambda i: (i, 0)),
          pl.BlockSpec((D, H), lambda i: (0, 0)),
          pl.BlockSpec((D, H), lambda i: (0, 0)),
      ],
      out_specs=[
          pl.BlockSpec((RB, H), lambda i: (i, 0)),
          pl.BlockSpec((RB, H), lambda i: (i, 0)),
      ],
      out_shape=[
          jax.ShapeDtypeStruct((N, H), jnp.float32),
          jax.ShapeDtypeStruct((N, H), jnp.float32),
      ],
  )(x, wl, wr)


def _tc_mid(parts0, cntt, r0, alpha, bb, wcat):
  def body(pp_ref, cn_ref, r0_ref, al_ref, bb_ref, w_ref, p1_ref, r1_ref):
    agg = pp_ref[0] + pp_ref[1]
    cnt = jnp.maximum(cn_ref[:, 0:1] + cn_ref[:, 1:2], 1.0)
    mean = agg / cnt
    h = jnp.maximum((mean + r0_ref[...]) * al_ref[...] + bb_ref[...], 0.0)
    pr = jnp.dot(h, w_ref[...], preferred_element_type=jnp.float32)
    p1_ref[...] = pr[:, :O]
    r1_ref[...] = pr[:, O:]

  return pl.pallas_call(
      body,
      grid=(N // RB,),
      in_specs=[
          pl.BlockSpec((NC, RB, H), lambda i: (0, i, 0)),
          pl.BlockSpec((RB, NC), lambda i: (i, 0)),
          pl.BlockSpec((RB, H), lambda i: (i, 0)),
          pl.BlockSpec((1, H), lambda i: (0, 0)),
          pl.BlockSpec((1, H), lambda i: (0, 0)),
          pl.BlockSpec((H, 2 * O), lambda i: (0, 0)),
      ],
      out_specs=[
          pl.BlockSpec((RB, O), lambda i: (i, 0)),
          pl.BlockSpec((RB, O), lambda i: (i, 0)),
      ],
      out_shape=[
          jax.ShapeDtypeStruct((N, O), jnp.float32),
          jax.ShapeDtypeStruct((N, O), jnp.float32),
      ],
  )(parts0, cntt, r0, alpha, bb, wcat)


def _tc_final(parts1, cntt, r1, b1):
  def body(pp_ref, cn_ref, r1_ref, b1_ref, out_ref):
    agg = pp_ref[0] + pp_ref[1]
    cnt = jnp.maximum(cn_ref[:, 0:1] + cn_ref[:, 1:2], 1.0)
    out_ref[...] = agg / cnt + r1_ref[...] + b1_ref[...]

  return pl.pallas_call(
      body,
      grid=(N // RB,),
      in_specs=[
          pl.BlockSpec((NC, RB, O), lambda i: (0, i, 0)),
          pl.BlockSpec((RB, NC), lambda i: (i, 0)),
          pl.BlockSpec((RB, O), lambda i: (i, 0)),
          pl.BlockSpec((1, O), lambda i: (0, 0)),
      ],
      out_specs=pl.BlockSpec((RB, O), lambda i: (i, 0)),
      out_shape=jax.ShapeDtypeStruct((N, O), jnp.float32),
  )(parts1, cntt, r1, b1)


def kernel(x, edge_index, Wl0, Wr0, b0, gamma0, beta0, Wl1, Wr1, b1):
  f32 = jnp.float32
  src = edge_index[0].reshape(NW, K, CH)
  dst = edge_index[1].reshape(NW, K, CH)
  ones = jnp.ones((CH,), f32)
  zvec = jnp.zeros((CB,), f32)
  zrows_h = jnp.zeros((CH, H), f32)
  zrows_o = jnp.zeros((CH, O), f32)

  p0, r0 = _tc_project(x, Wl0, Wr0)
  parts0, cntp = _seg_sum_cnt(p0, src, dst, ones, zrows_h, zvec)
  cntt = cntp.T

  scale = 1.0 / jnp.sqrt(jnp.float32(1.0) + BN_EPS)
  alpha = (gamma0 * scale).reshape(1, H)
  bb = (b0 * gamma0 * scale + beta0).reshape(1, H)
  wcat = jnp.concatenate([Wl1, Wr1], axis=1)

  p1, r1 = _tc_mid(parts0, cntt, r0, alpha, bb, wcat)
  (parts1,) = _seg_sum_o(p1, src, dst, ones, zrows_o, zvec)
  out = _tc_final(parts1, cntt, r1, b1.reshape(1, O))
  return out


# speedup vs baseline: 1.0430x; 1.0430x over previous
import functools

import jax
import jax.numpy as jnp
from jax import lax
from jax.experimental import pallas as pl
from jax.experimental.pallas import tpu as pltpu
from jax.experimental.pallas import tpu_sc as plsc

N = 10000
E = 320000
D = 128
H = 64
O = 32
BN_EPS = 1e-5

NC, NS = 2, 16
NW = NC * NS
CH = 125
K = E // (NW * CH)
RPW = N // NS
CB = 1000
RB = 2000


def _make_seg_sum(width: int, with_cnt: bool, nb: int):
  mesh = plsc.VectorSubcoreMesh(core_axis_name="c", subcore_axis_name="s")
  out_type = [jax.ShapeDtypeStruct((NC, N, width), jnp.float32)]
  if with_cnt:
    out_type.append(jax.ShapeDtypeStruct((NC, N), jnp.float32))
  scratch = [
      pltpu.VMEM((K, CH), jnp.int32),
      pltpu.VMEM((K, CH), jnp.int32),
  ] + [pltpu.VMEM((CH, width), jnp.float32) for _ in range(nb)]
  if with_cnt:
    scratch += [
        pltpu.VMEM((CH,), jnp.float32),
        pltpu.VMEM((CB,), jnp.float32),
    ]
  scratch += [
      pltpu.VMEM_SHARED((N, width), jnp.float32),
      pltpu.VMEM_SHARED((N, width), jnp.float32),
  ]
  if with_cnt:
    scratch += [pltpu.VMEM_SHARED((N,), jnp.float32)]
  scratch += [pltpu.SemaphoreType.DMA for _ in range(2 * nb)]

  def body(table, edges, ones, zrows, zvec, *rest):
    if with_cnt:
      (parts, cnts, src_v, dst_v, *rest2) = rest
      rows = tuple(rest2[:nb])
      ones_v, zv_v, acc, tbl_sh, cacc = rest2[nb:nb + 5]
      sems = rest2[nb + 5:]
    else:
      (parts, src_v, dst_v, *rest2) = rest
      cnts = None
      rows = tuple(rest2[:nb])
      acc, tbl_sh = rest2[nb:nb + 2]
      ones_v = zv_v = cacc = None
      sems = rest2[nb + 2:]
    gsems = tuple(sems[:nb])
    ssems = tuple(sems[nb:2 * nb])
    rows0 = rows[0]
    sid = lax.axis_index("s")
    cid = lax.axis_index("c")
    wid = sid * NC + cid
    r0 = sid * RPW

    pltpu.sync_copy(zrows, rows0)
    if with_cnt:
      pltpu.sync_copy(zvec, zv_v)
    for t in range(RPW // CH):
      base = r0 + t * CH
      pltpu.sync_copy(rows0, acc.at[pl.ds(base, CH)])
    if with_cnt:
      @pl.when(sid < N // CB)
      def _zero_cnt():
        pltpu.sync_copy(zv_v, cacc.at[pl.ds(sid * CB, CB)])
      pltpu.sync_copy(ones, ones_v)
    pltpu.sync_copy(edges.at[0, wid], src_v)
    pltpu.sync_copy(edges.at[1, wid], dst_v)
    pltpu.sync_copy(table.at[pl.ds(r0, RPW)], tbl_sh.at[pl.ds(r0, RPW)])
    plsc.subcore_barrier()

    pltpu.async_copy(tbl_sh.at[src_v.at[0]], rows[0], gsems[0])
    pltpu.async_copy(tbl_sh.at[src_v.at[1]], rows[1], gsems[1])

    @pl.loop(0, K, step=nb)
    def _edge_ring(j):
      for b in range(nb):
        jj = j + b
        b2 = (b + 2) % nb
        pltpu.make_async_copy(tbl_sh.at[src_v.at[jj]], rows[b], gsems[b]).wait()
        pltpu.async_copy(rows[b], acc.at[dst_v.at[jj]], ssems[b], add=True)
        if with_cnt:
          pltpu.async_copy(ones_v, cacc.at[dst_v.at[jj]], ssems[b], add=True)

        @pl.when(jj + 2 < K)
        def _prefetch():
          prev = jj + 2 - nb

          @pl.when(prev >= 0)
          def _drain_scatter():
            pltpu.make_async_copy(
                rows[b2], acc.at[dst_v.at[prev]], ssems[b2]).wait()
            if with_cnt:
              pltpu.make_async_copy(
                  ones_v, cacc.at[dst_v.at[prev]], ssems[b2]).wait()
          pltpu.async_copy(tbl_sh.at[src_v.at[jj + 2]], rows[b2], gsems[b2])

    for c in range(K - nb, K):
      b = c % nb
      pltpu.make_async_copy(rows[b], acc.at[dst_v.at[c]], ssems[b]).wait()
      if with_cnt:
        pltpu.make_async_copy(ones_v, cacc.at[dst_v.at[c]], ssems[b]).wait()

    plsc.subcore_barrier()

    for t in range(RPW // CH):
      base = r0 + t * CH
      pltpu.sync_copy(acc.at[pl.ds(base, CH)], rows0)
      pltpu.sync_copy(rows0, parts.at[cid, pl.ds(base, CH)])
    if with_cnt:
      @pl.when(sid < N // CB)
      def _write_cnt():
        pltpu.sync_copy(cacc.at[pl.ds(sid * CB, CB)], zv_v)
        pltpu.sync_copy(zv_v, cnts.at[cid, pl.ds(sid * CB, CB)])

  return pl.kernel(body, out_type=tuple(out_type), mesh=mesh,
                   scratch_types=scratch,
                   compiler_params=pltpu.CompilerParams(
                       use_tc_tiling_on_sc=False))


_seg_sum_cnt = _make_seg_sum(H, with_cnt=True, nb=2)
_seg_sum_o = _make_seg_sum(O, with_cnt=False, nb=2)


def _tc_matmul(x, w, ow):
  def body(x_ref, w_ref, o_ref):
    o_ref[...] = jnp.dot(x_ref[...], w_ref[...],
                         preferred_element_type=jnp.float32)

  return pl.pallas_call(
      body,
      grid=(N // RB,),
      in_specs=[
          pl.BlockSpec((RB, x.shape[1]), lambda i: (i, 0)),
          pl.BlockSpec((x.shape[1], ow), lambda i: (0, 0)),
      ],
      out_specs=pl.BlockSpec((RB, ow), lambda i: (i, 0)),
      out_shape=jax.ShapeDtypeStruct((N, ow), jnp.float32),
  )(x, w)


def _tc_mid(parts0, cntt, r0, alpha, bb, w, ow):
  def body(pp_ref, cn_ref, r0_ref, al_ref, bb_ref, w_ref, o_ref):
    agg = pp_ref[0] + pp_ref[1]
    cnt = jnp.maximum(cn_ref[:, 0:1] + cn_ref[:, 1:2], 1.0)
    mean = agg / cnt
    h = jnp.maximum((mean + r0_ref[...]) * al_ref[...] + bb_ref[...], 0.0)
    o_ref[...] = jnp.dot(h, w_ref[...], preferred_element_type=jnp.float32)

  return pl.pallas_call(
      body,
      grid=(N // RB,),
      in_specs=[
          pl.BlockSpec((NC, RB, H), lambda i: (0, i, 0)),
          pl.BlockSpec((RB, NC), lambda i: (i, 0)),
          pl.BlockSpec((RB, H), lambda i: (i, 0)),
          pl.BlockSpec((1, H), lambda i: (0, 0)),
          pl.BlockSpec((1, H), lambda i: (0, 0)),
          pl.BlockSpec((H, ow), lambda i: (0, 0)),
      ],
      out_specs=pl.BlockSpec((RB, ow), lambda i: (i, 0)),
      out_shape=jax.ShapeDtypeStruct((N, ow), jnp.float32),
  )(parts0, cntt, r0, alpha, bb, w)


def _tc_final(parts1, cntt, r1, b1):
  def body(pp_ref, cn_ref, r1_ref, b1_ref, out_ref):
    agg = pp_ref[0] + pp_ref[1]
    cnt = jnp.maximum(cn_ref[:, 0:1] + cn_ref[:, 1:2], 1.0)
    out_ref[...] = agg / cnt + r1_ref[...] + b1_ref[...]

  return pl.pallas_call(
      body,
      grid=(N // RB,),
      in_specs=[
          pl.BlockSpec((NC, RB, O), lambda i: (0, i, 0)),
          pl.BlockSpec((RB, NC), lambda i: (i, 0)),
          pl.BlockSpec((RB, O), lambda i: (i, 0)),
          pl.BlockSpec((1, O), lambda i: (0, 0)),
      ],
      out_specs=pl.BlockSpec((RB, O), lambda i: (i, 0)),
      out_shape=jax.ShapeDtypeStruct((N, O), jnp.float32),
  )(parts1, cntt, r1, b1)


def kernel(x, edge_index, Wl0, Wr0, b0, gamma0, beta0, Wl1, Wr1, b1):
  f32 = jnp.float32
  edges = edge_index.reshape(2, NW, K, CH)
  ones = jnp.ones((CH,), f32)
  zvec = jnp.zeros((CB,), f32)
  zrows_h = jnp.zeros((CH, H), f32)
  zrows_o = jnp.zeros((CH, O), f32)

  p0 = _tc_matmul(x, Wl0, H)
  parts0, cntp = _seg_sum_cnt(p0, edges, ones, zrows_h, zvec)
  r0 = _tc_matmul(x, Wr0, H)
  cntt = cntp.T

  scale = 1.0 / jnp.sqrt(jnp.float32(1.0) + BN_EPS)
  alpha = (gamma0 * scale).reshape(1, H)
  bb = (b0 * gamma0 * scale + beta0).reshape(1, H)

  p1 = _tc_mid(parts0, cntt, r0, alpha, bb, Wl1, O)
  (parts1,) = _seg_sum_o(p1, edges, ones, zrows_o, zvec)
  r1 = _tc_mid(parts0, cntt, r0, alpha, bb, Wr1, O)
  out = _tc_final(parts1, cntt, r1, b1.reshape(1, O))
  return out


# speedup vs baseline: 1.0932x; 1.0481x over previous
import functools

import jax
import jax.numpy as jnp
from jax import lax
from jax.experimental import pallas as pl
from jax.experimental.pallas import tpu as pltpu
from jax.experimental.pallas import tpu_sc as plsc

N = 10000
E = 320000
D = 128
H = 64
O = 32
BN_EPS = 1e-5

NC, NS = 2, 16
NW = NC * NS
CH = 125
K = E // (NW * CH)
RPW = N // NS
CB = 1000
RB = 2048
GRID = (N + RB - 1) // RB


def _make_seg_sum(width: int, with_cnt: bool, nb: int):
  mesh = plsc.VectorSubcoreMesh(core_axis_name="c", subcore_axis_name="s")
  out_type = [jax.ShapeDtypeStruct((NC, N, width), jnp.float32)]
  if with_cnt:
    out_type.append(jax.ShapeDtypeStruct((NC, N), jnp.float32))
  scratch = [
      pltpu.VMEM((K, CH), jnp.int32),
      pltpu.VMEM((K, CH), jnp.int32),
  ] + [pltpu.VMEM((CH, width), jnp.float32) for _ in range(nb)]
  if with_cnt:
    scratch += [
        pltpu.VMEM((CH,), jnp.float32),
        pltpu.VMEM((CB,), jnp.float32),
    ]
  scratch += [
      pltpu.VMEM_SHARED((N, width), jnp.float32),
      pltpu.VMEM_SHARED((N, width), jnp.float32),
  ]
  if with_cnt:
    scratch += [pltpu.VMEM_SHARED((N,), jnp.float32)]
  scratch += [pltpu.SemaphoreType.DMA for _ in range(2 * nb)]

  def body(table, edges, ones, zrows, zvec, *rest):
    if with_cnt:
      (parts, cnts, src_v, dst_v, *rest2) = rest
      rows = tuple(rest2[:nb])
      ones_v, zv_v, acc, tbl_sh, cacc = rest2[nb:nb + 5]
      sems = rest2[nb + 5:]
    else:
      (parts, src_v, dst_v, *rest2) = rest
      cnts = None
      rows = tuple(rest2[:nb])
      acc, tbl_sh = rest2[nb:nb + 2]
      ones_v = zv_v = cacc = None
      sems = rest2[nb + 2:]
    gsems = tuple(sems[:nb])
    ssems = tuple(sems[nb:2 * nb])
    rows0 = rows[0]
    sid = lax.axis_index("s")
    cid = lax.axis_index("c")
    wid = sid * NC + cid
    r0 = sid * RPW

    pltpu.sync_copy(zrows, rows0)
    if with_cnt:
      pltpu.sync_copy(zvec, zv_v)
    for t in range(RPW // CH):
      base = r0 + t * CH
      pltpu.sync_copy(rows0, acc.at[pl.ds(base, CH)])
    if with_cnt:
      @pl.when(sid < N // CB)
      def _zero_cnt():
        pltpu.sync_copy(zv_v, cacc.at[pl.ds(sid * CB, CB)])
      pltpu.sync_copy(ones, ones_v)
    pltpu.sync_copy(edges.at[0, wid], src_v)
    pltpu.sync_copy(edges.at[1, wid], dst_v)
    pltpu.sync_copy(table.at[pl.ds(r0, RPW)], tbl_sh.at[pl.ds(r0, RPW)])
    plsc.subcore_barrier()

    pltpu.async_copy(tbl_sh.at[src_v.at[0]], rows[0], gsems[0])
    pltpu.async_copy(tbl_sh.at[src_v.at[1]], rows[1], gsems[1])

    @pl.loop(0, K, step=nb)
    def _edge_ring(j):
      for b in range(nb):
        jj = j + b
        b2 = (b + 2) % nb
        pltpu.make_async_copy(tbl_sh.at[src_v.at[jj]], rows[b], gsems[b]).wait()
        pltpu.async_copy(rows[b], acc.at[dst_v.at[jj]], ssems[b], add=True)
        if with_cnt:
          pltpu.async_copy(ones_v, cacc.at[dst_v.at[jj]], ssems[b], add=True)

        @pl.when(jj + 2 < K)
        def _prefetch():
          prev = jj + 2 - nb

          @pl.when(prev >= 0)
          def _drain_scatter():
            pltpu.make_async_copy(
                rows[b2], acc.at[dst_v.at[prev]], ssems[b2]).wait()
            if with_cnt:
              pltpu.make_async_copy(
                  ones_v, cacc.at[dst_v.at[prev]], ssems[b2]).wait()
          pltpu.async_copy(tbl_sh.at[src_v.at[jj + 2]], rows[b2], gsems[b2])

    for c in range(K - nb, K):
      b = c % nb
      pltpu.make_async_copy(rows[b], acc.at[dst_v.at[c]], ssems[b]).wait()
      if with_cnt:
        pltpu.make_async_copy(ones_v, cacc.at[dst_v.at[c]], ssems[b]).wait()

    plsc.subcore_barrier()

    for t in range(RPW // CH):
      base = r0 + t * CH
      pltpu.sync_copy(acc.at[pl.ds(base, CH)], rows0)
      pltpu.sync_copy(rows0, parts.at[cid, pl.ds(base, CH)])
    if with_cnt:
      @pl.when(sid < N // CB)
      def _write_cnt():
        pltpu.sync_copy(cacc.at[pl.ds(sid * CB, CB)], zv_v)
        pltpu.sync_copy(zv_v, cnts.at[cid, pl.ds(sid * CB, CB)])

  return pl.kernel(body, out_type=tuple(out_type), mesh=mesh,
                   scratch_types=scratch,
                   compiler_params=pltpu.CompilerParams(
                       use_tc_tiling_on_sc=False))


_seg_sum_cnt = _make_seg_sum(H, with_cnt=True, nb=2)
_seg_sum_o = _make_seg_sum(O, with_cnt=False, nb=2)


def _tc_matmul(x, w, ow):
  def body(x_ref, w_ref, o_ref):
    o_ref[...] = jnp.dot(x_ref[...], w_ref[...],
                         preferred_element_type=jnp.float32)

  return pl.pallas_call(
      body,
      grid=(GRID,),
      in_specs=[
          pl.BlockSpec((RB, x.shape[1]), lambda i: (i, 0)),
          pl.BlockSpec((x.shape[1], ow), lambda i: (0, 0)),
      ],
      out_specs=pl.BlockSpec((RB, ow), lambda i: (i, 0)),
      out_shape=jax.ShapeDtypeStruct((N, ow), jnp.float32),
  )(x, w)


ND = N * O // 128
RBD = RB * O // 128


def _tc_mid(parts0, cntt, r0, alpha, bb, w):
  def body(pp_ref, cn_ref, r0_ref, al_ref, bb_ref, w_ref, o_ref):
    agg = pp_ref[0] + pp_ref[1]
    cnt = jnp.maximum(cn_ref[:, 0:1] + cn_ref[:, 1:2], 1.0)
    mean = agg / cnt
    h = jnp.maximum((mean + r0_ref[...]) * al_ref[...] + bb_ref[...], 0.0)
    o_ref[...] = jnp.dot(h, w_ref[...], preferred_element_type=jnp.float32)

  return pl.pallas_call(
      body,
      grid=(GRID,),
      in_specs=[
          pl.BlockSpec((NC, RB, H), lambda i: (0, i, 0)),
          pl.BlockSpec((RB, NC), lambda i: (i, 0)),
          pl.BlockSpec((RB, H), lambda i: (i, 0)),
          pl.BlockSpec((1, H), lambda i: (0, 0)),
          pl.BlockSpec((1, H), lambda i: (0, 0)),
          pl.BlockSpec((H, O), lambda i: (0, 0)),
      ],
      out_specs=pl.BlockSpec((RB, O), lambda i: (i, 0)),
      out_shape=jax.ShapeDtypeStruct((N, O), jnp.float32),
  )(parts0, cntt, r0, alpha, bb, w)


def _tc_final(parts1r, cntb, r1d, b1t):
  def body(pp_ref, cb_ref, r1_ref, b1_ref, out_ref):
    agg = pp_ref[0] + pp_ref[1]
    out_ref[...] = agg / cb_ref[...] + r1_ref[...] + b1_ref[...]

  return pl.pallas_call(
      body,
      grid=(GRID,),
      in_specs=[
          pl.BlockSpec((NC, RBD, 128), lambda i: (0, i, 0)),
          pl.BlockSpec((RBD, 128), lambda i: (i, 0)),
          pl.BlockSpec((RBD, 128), lambda i: (i, 0)),
          pl.BlockSpec((1, 128), lambda i: (0, 0)),
      ],
      out_specs=pl.BlockSpec((RBD, 128), lambda i: (i, 0)),
      out_shape=jax.ShapeDtypeStruct((ND, 128), jnp.float32),
  )(parts1r, cntb, r1d, b1t)


def kernel(x, edge_index, Wl0, Wr0, b0, gamma0, beta0, Wl1, Wr1, b1):
  f32 = jnp.float32
  edges = edge_index.reshape(2, NW, K, CH)
  ones = jnp.ones((CH,), f32)
  zvec = jnp.zeros((CB,), f32)
  zrows_h = jnp.zeros((CH, H), f32)
  zrows_o = jnp.zeros((CH, O), f32)

  p0 = _tc_matmul(x, Wl0, H)
  parts0, cntp = _seg_sum_cnt(p0, edges, ones, zrows_h, zvec)
  r0 = _tc_matmul(x, Wr0, H)
  cntt = cntp.T

  scale = 1.0 / jnp.sqrt(jnp.float32(1.0) + BN_EPS)
  alpha = (gamma0 * scale).reshape(1, H)
  bb = (b0 * gamma0 * scale + beta0).reshape(1, H)

  p1 = _tc_mid(parts0, cntt, r0, alpha, bb, Wl1)
  (parts1,) = _seg_sum_o(p1, edges, ones, zrows_o, zvec)
  r1 = _tc_mid(parts0, cntt, r0, alpha, bb, Wr1)
  cntb = jnp.repeat(jnp.maximum(cntp[0] + cntp[1], 1.0), O).reshape(ND, 128)
  parts1r = parts1.reshape(NC, ND, 128)
  r1d = r1.reshape(ND, 128)
  b1t = jnp.tile(b1.reshape(1, O), (1, 128 // O))
  out = _tc_final(parts1r, cntb, r1d, b1t)
  return out.reshape(N, O)


# speedup vs baseline: 1.1538x; 1.0555x over previous
import functools

import jax
import jax.numpy as jnp
from jax import lax
from jax.experimental import pallas as pl
from jax.experimental.pallas import tpu as pltpu
from jax.experimental.pallas import tpu_sc as plsc

N = 10000
E = 320000
D = 128
H = 64
O = 32
BN_EPS = 1e-5

NC, NS = 2, 16
NW = NC * NS
CH = 125
K = E // (NW * CH)
RPW = N // NS
CB = 1000
RB = 2048
GRID = (N + RB - 1) // RB


def _make_seg_sum(width: int, with_cnt: bool, nb: int):
  mesh = plsc.VectorSubcoreMesh(core_axis_name="c", subcore_axis_name="s")
  out_type = [jax.ShapeDtypeStruct((NC, N, width), jnp.float32)]
  if with_cnt:
    out_type.append(jax.ShapeDtypeStruct((NC, N), jnp.float32))
  scratch = [
      pltpu.VMEM((K, CH), jnp.int32),
      pltpu.VMEM((K, CH), jnp.int32),
  ] + [pltpu.VMEM((CH, width), jnp.float32) for _ in range(nb)]
  if with_cnt:
    scratch += [
        pltpu.VMEM((CH,), jnp.float32),
        pltpu.VMEM((CB,), jnp.float32),
    ]
  scratch += [
      pltpu.VMEM_SHARED((N, width), jnp.float32),
      pltpu.VMEM_SHARED((N, width), jnp.float32),
  ]
  if with_cnt:
    scratch += [pltpu.VMEM_SHARED((N,), jnp.float32)]
  scratch += [pltpu.SemaphoreType.DMA for _ in range(2 * nb)]

  def body(table, edges, ones, zrows, zvec, *rest):
    if with_cnt:
      (parts, cnts, src_v, dst_v, *rest2) = rest
      rows = tuple(rest2[:nb])
      ones_v, zv_v, acc, tbl_sh, cacc = rest2[nb:nb + 5]
      sems = rest2[nb + 5:]
    else:
      (parts, src_v, dst_v, *rest2) = rest
      cnts = None
      rows = tuple(rest2[:nb])
      acc, tbl_sh = rest2[nb:nb + 2]
      ones_v = zv_v = cacc = None
      sems = rest2[nb + 2:]
    gsems = tuple(sems[:nb])
    ssems = tuple(sems[nb:2 * nb])
    rows0 = rows[0]
    sid = lax.axis_index("s")
    cid = lax.axis_index("c")
    wid = sid * NC + cid
    r0 = sid * RPW

    pltpu.sync_copy(zrows, rows0)
    if with_cnt:
      pltpu.sync_copy(zvec, zv_v)
    for t in range(RPW // CH):
      base = r0 + t * CH
      pltpu.sync_copy(rows0, acc.at[pl.ds(base, CH)])
    if with_cnt:
      @pl.when(sid < N // CB)
      def _zero_cnt():
        pltpu.sync_copy(zv_v, cacc.at[pl.ds(sid * CB, CB)])
      pltpu.sync_copy(ones, ones_v)
    pltpu.sync_copy(edges.at[0, wid], src_v)
    pltpu.sync_copy(edges.at[1, wid], dst_v)
    pltpu.sync_copy(table.at[pl.ds(r0, RPW)], tbl_sh.at[pl.ds(r0, RPW)])
    plsc.subcore_barrier()

    pltpu.async_copy(tbl_sh.at[src_v.at[0]], rows[0], gsems[0])
    pltpu.async_copy(tbl_sh.at[src_v.at[1]], rows[1], gsems[1])

    @pl.loop(0, K, step=nb)
    def _edge_ring(j):
      for b in range(nb):
        jj = j + b
        b2 = (b + 2) % nb
        pltpu.make_async_copy(tbl_sh.at[src_v.at[jj]], rows[b], gsems[b]).wait()
        pltpu.async_copy(rows[b], acc.at[dst_v.at[jj]], ssems[b], add=True)
        if with_cnt:
          pltpu.async_copy(ones_v, cacc.at[dst_v.at[jj]], ssems[b], add=True)

        @pl.when(jj + 2 < K)
        def _prefetch():
          prev = jj + 2 - nb

          @pl.when(prev >= 0)
          def _drain_scatter():
            pltpu.make_async_copy(
                rows[b2], acc.at[dst_v.at[prev]], ssems[b2]).wait()
            if with_cnt:
              pltpu.make_async_copy(
                  ones_v, cacc.at[dst_v.at[prev]], ssems[b2]).wait()
          pltpu.async_copy(tbl_sh.at[src_v.at[jj + 2]], rows[b2], gsems[b2])

    for c in range(K - nb, K):
      b = c % nb
      pltpu.make_async_copy(rows[b], acc.at[dst_v.at[c]], ssems[b]).wait()
      if with_cnt:
        pltpu.make_async_copy(ones_v, cacc.at[dst_v.at[c]], ssems[b]).wait()

    plsc.subcore_barrier()

    for t in range(RPW // CH):
      base = r0 + t * CH
      pltpu.sync_copy(acc.at[pl.ds(base, CH)], rows0)
      pltpu.sync_copy(rows0, parts.at[cid, pl.ds(base, CH)])
    if with_cnt:
      @pl.when(sid < N // CB)
      def _write_cnt():
        pltpu.sync_copy(cacc.at[pl.ds(sid * CB, CB)], zv_v)
        pltpu.sync_copy(zv_v, cnts.at[cid, pl.ds(sid * CB, CB)])

  return pl.kernel(body, out_type=tuple(out_type), mesh=mesh,
                   scratch_types=scratch,
                   compiler_params=pltpu.CompilerParams(
                       use_tc_tiling_on_sc=False))


_seg_sum_cnt = _make_seg_sum(H, with_cnt=True, nb=2)
_seg_sum_o = _make_seg_sum(O, with_cnt=False, nb=2)


def _tc_matmul(x, w, ow):
  def body(x_ref, w_ref, o_ref):
    o_ref[...] = jnp.dot(x_ref[...], w_ref[...],
                         preferred_element_type=jnp.float32)

  return pl.pallas_call(
      body,
      grid=(GRID,),
      in_specs=[
          pl.BlockSpec((RB, x.shape[1]), lambda i: (i, 0)),
          pl.BlockSpec((x.shape[1], ow), lambda i: (0, 0)),
      ],
      out_specs=pl.BlockSpec((RB, ow), lambda i: (i, 0)),
      out_shape=jax.ShapeDtypeStruct((N, ow), jnp.float32),
  )(x, w)


ND = N * O // 128
RBD = RB * O // 128


NDH = N * H // 128
RBD2 = RB // 2


def _tc_mid(parts0d, cntb0, r0d, alpha2, bb2, w2):
  def body(pp_ref, cb_ref, r0_ref, al_ref, bb_ref, w_ref, o_ref):
    agg = pp_ref[0] + pp_ref[1]
    h = jnp.maximum(
        (agg / cb_ref[...] + r0_ref[...]) * al_ref[...] + bb_ref[...], 0.0)
    o_ref[...] = jnp.dot(h, w_ref[...], preferred_element_type=jnp.float32)

  return pl.pallas_call(
      body,
      grid=(GRID,),
      in_specs=[
          pl.BlockSpec((NC, RBD2, 128), lambda i: (0, i, 0)),
          pl.BlockSpec((RBD2, 128), lambda i: (i, 0)),
          pl.BlockSpec((RBD2, 128), lambda i: (i, 0)),
          pl.BlockSpec((1, 128), lambda i: (0, 0)),
          pl.BlockSpec((1, 128), lambda i: (0, 0)),
          pl.BlockSpec((128, 2 * O), lambda i: (0, 0)),
      ],
      out_specs=pl.BlockSpec((RBD2, 2 * O), lambda i: (i, 0)),
      out_shape=jax.ShapeDtypeStruct((NDH, 2 * O), jnp.float32),
  )(parts0d, cntb0, r0d, alpha2, bb2, w2)


def _tc_final(parts1r, cntb, r1d, b1t):
  def body(pp_ref, cb_ref, r1_ref, b1_ref, out_ref):
    agg = pp_ref[0] + pp_ref[1]
    out_ref[...] = agg / cb_ref[...] + r1_ref[...] + b1_ref[...]

  return pl.pallas_call(
      body,
      grid=(GRID,),
      in_specs=[
          pl.BlockSpec((NC, RBD, 128), lambda i: (0, i, 0)),
          pl.BlockSpec((RBD, 128), lambda i: (i, 0)),
          pl.BlockSpec((RBD, 128), lambda i: (i, 0)),
          pl.BlockSpec((1, 128), lambda i: (0, 0)),
      ],
      out_specs=pl.BlockSpec((RBD, 128), lambda i: (i, 0)),
      out_shape=jax.ShapeDtypeStruct((ND, 128), jnp.float32),
  )(parts1r, cntb, r1d, b1t)


def kernel(x, edge_index, Wl0, Wr0, b0, gamma0, beta0, Wl1, Wr1, b1):
  f32 = jnp.float32
  edges = edge_index.reshape(2, NW, K, CH)
  ones = jnp.ones((CH,), f32)
  zvec = jnp.zeros((CB,), f32)
  zrows_h = jnp.zeros((CH, H), f32)
  zrows_o = jnp.zeros((CH, O), f32)

  p0 = _tc_matmul(x, Wl0, H)
  parts0, cntp = _seg_sum_cnt(p0, edges, ones, zrows_h, zvec)
  r0 = _tc_matmul(x, Wr0, H)

  scale = 1.0 / jnp.sqrt(jnp.float32(1.0) + BN_EPS)
  alpha = (gamma0 * scale).reshape(1, H)
  bb = (b0 * gamma0 * scale + beta0).reshape(1, H)

  cntc = jnp.maximum(cntp[0] + cntp[1], 1.0)
  cntb0 = jnp.repeat(cntc, H).reshape(NDH, 128)
  parts0d = parts0.reshape(NC, NDH, 128)
  r0d = r0.reshape(NDH, 128)
  alpha2 = jnp.tile(alpha, (1, 2))
  bb2 = jnp.tile(bb, (1, 2))
  z = jnp.zeros((H, O), jnp.float32)
  w2l = jnp.concatenate(
      [jnp.concatenate([Wl1, z], 1), jnp.concatenate([z, Wl1], 1)], 0)
  w2r = jnp.concatenate(
      [jnp.concatenate([Wr1, z], 1), jnp.concatenate([z, Wr1], 1)], 0)

  p1 = _tc_mid(parts0d, cntb0, r0d, alpha2, bb2, w2l)
  (parts1,) = _seg_sum_o(p1.reshape(N, O), edges, ones, zrows_o, zvec)
  r1 = _tc_mid(parts0d, cntb0, r0d, alpha2, bb2, w2r)
  cntb = jnp.repeat(cntc, O).reshape(ND, 128)
  parts1r = parts1.reshape(NC, ND, 128)
  r1d = r1.reshape(ND, 128)
  b1t = jnp.tile(b1.reshape(1, O), (1, 128 // O))
  out = _tc_final(parts1r, cntb, r1d, b1t)
  return out.reshape(N, O)
